# baseline (speedup 1.0000x reference)
import jax
import jax.numpy as jnp
from jax import lax
from jax.experimental import pallas as pl
from jax.experimental.pallas import tpu as pltpu
from jax.experimental.pallas import tpu_sc as plsc

EMB_SIZE = 100000
EMB_DIM = 64
D_HALF = EMB_DIM // 2
BATCH = 16384

NUM_CORES = 2
NUM_SUBCORES = 16
NUM_WORKERS = NUM_CORES * NUM_SUBCORES
B_PER_W = BATCH // NUM_WORKERS
IDX_CHUNK = 64
N_CHUNKS = B_PER_W // IDX_CHUNK
GROUPS_PER_CHUNK = IDX_CHUNK // 16

_SC_PARAMS = pltpu.CompilerParams(use_tc_tiling_on_sc=False,
                                  needs_layout_passes=False)
_MESH = plsc.VectorSubcoreMesh(core_axis_name="c", subcore_axis_name="s")


def _worker_base():
    wid = lax.axis_index("s") * NUM_CORES + lax.axis_index("c")
    return wid * B_PER_W


def _dot16(rows1, rows2, base_elem, lane):
    row_idx = base_elem + lane
    d_vec = lane
    acc = jnp.zeros((16,), jnp.float32)
    for _ in range(D_HALF):
        a = plsc.load_gather(rows1, [row_idx, d_vec])
        b = plsc.load_gather(rows2, [row_idx, d_vec])
        acc = acc + a * b
        d_vec = (d_vec + 1) % D_HALF
    return acc


def _bias_body(p1_hbm, p2_hbm, bias_hbm, b0_hbm, out_hbm,
               idx1_v, idx2_v, bv1_v, bv2_v, b0_v, bs_v, sem):
    base = _worker_base()
    pltpu.sync_copy(p1_hbm.at[pl.ds(base, B_PER_W)], idx1_v)
    pltpu.sync_copy(p2_hbm.at[pl.ds(base, B_PER_W)], idx2_v)
    pltpu.sync_copy(b0_hbm, b0_v)
    cb1 = pltpu.async_copy(bias_hbm.at[idx1_v], bv1_v, sem)
    cb2 = pltpu.async_copy(bias_hbm.at[idx2_v], bv2_v, sem)
    cb1.wait()
    cb2.wait()
    b0 = b0_v[...]

    def body(g, _):
        e0 = g * 16
        bs_v[pl.ds(e0, 16)] = bv1_v[pl.ds(e0, 16)] + bv2_v[pl.ds(e0, 16)] + b0
        return 0

    lax.fori_loop(0, B_PER_W // 16, body, 0)
    pltpu.sync_copy(bs_v, out_hbm.at[pl.ds(base, B_PER_W)])


_sc_bias = pl.kernel(
    _bias_body,
    out_type=jax.ShapeDtypeStruct((BATCH,), jnp.float32),
    mesh=_MESH,
    scratch_types=[
        pltpu.VMEM((B_PER_W,), jnp.int32),
        pltpu.VMEM((B_PER_W,), jnp.int32),
        pltpu.VMEM((B_PER_W,), jnp.float32),
        pltpu.VMEM((B_PER_W,), jnp.float32),
        pltpu.VMEM((16,), jnp.float32),
        pltpu.VMEM((B_PER_W,), jnp.float32),
        pltpu.SemaphoreType.DMA,
    ],
    compiler_params=_SC_PARAMS,
)


def _gather_chunks(emb_hbm, idx1_v, idx2_v, rows1_v, rows2_v, sems):
    copies = []
    for c in range(N_CHUNKS):
        sl = pl.ds(c * IDX_CHUNK, IDX_CHUNK)
        copies.append((
            pltpu.async_copy(emb_hbm.at[idx1_v.at[sl]], rows1_v.at[sl],
                             sems[c]),
            pltpu.async_copy(emb_hbm.at[idx2_v.at[sl]], rows2_v.at[sl],
                             sems[c]),
        ))
    return copies


def _half0_body(p1_hbm, p2_hbm, emb_hbm, out_hbm,
                idx1_v, idx2_v, rows1_v, rows2_v, out_v, *sems):
    base = _worker_base()
    lane = lax.iota(jnp.int32, 16)
    pltpu.sync_copy(p1_hbm.at[pl.ds(base, B_PER_W)], idx1_v)
    pltpu.sync_copy(p2_hbm.at[pl.ds(base, B_PER_W)], idx2_v)
    copies = _gather_chunks(emb_hbm, idx1_v, idx2_v, rows1_v, rows2_v, sems)

    for c in range(N_CHUNKS):
        copies[c][0].wait()
        copies[c][1].wait()

        def group_body(g, _):
            e0 = c * IDX_CHUNK + g * 16
            out_v[pl.ds(e0, 16)] = _dot16(rows1_v, rows2_v, e0, lane)
            return 0

        lax.fori_loop(0, GROUPS_PER_CHUNK, group_body, 0)

    pltpu.sync_copy(out_v, out_hbm.at[pl.ds(base, B_PER_W)])


def _half1_body(p1_hbm, p2_hbm, emb_hbm, part_hbm, bsum_hbm, out_hbm,
                idx1_v, idx2_v, rows1_v, rows2_v, part_v, bsum_v, out_v,
                sem_p, *sems):
    base = _worker_base()
    lane = lax.iota(jnp.int32, 16)
    pltpu.sync_copy(p1_hbm.at[pl.ds(base, B_PER_W)], idx1_v)
    pltpu.sync_copy(p2_hbm.at[pl.ds(base, B_PER_W)], idx2_v)
    copies = _gather_chunks(emb_hbm, idx1_v, idx2_v, rows1_v, rows2_v, sems)
    cp = pltpu.async_copy(part_hbm.at[pl.ds(base, B_PER_W)], part_v, sem_p)
    cb = pltpu.async_copy(bsum_hbm.at[pl.ds(base, B_PER_W)], bsum_v, sem_p)

    for c in range(N_CHUNKS):
        copies[c][0].wait()
        copies[c][1].wait()

        def group_body(g, _):
            e0 = c * IDX_CHUNK + g * 16
            out_v[pl.ds(e0, 16)] = _dot16(rows1_v, rows2_v, e0, lane)
            return 0

        lax.fori_loop(0, GROUPS_PER_CHUNK, group_body, 0)

    cp.wait()
    cb.wait()

    def fin_body(g, _):
        e0 = g * 16
        z = out_v[pl.ds(e0, 16)] + part_v[pl.ds(e0, 16)] \
            + bsum_v[pl.ds(e0, 16)]
        out_v[pl.ds(e0, 16)] = 1.0 / (1.0 + jnp.exp(-z))
        return 0

    lax.fori_loop(0, B_PER_W // 16, fin_body, 0)
    pltpu.sync_copy(out_v, out_hbm.at[pl.ds(base, B_PER_W)])


_ROWS_SCRATCH = [
    pltpu.VMEM((B_PER_W,), jnp.int32),
    pltpu.VMEM((B_PER_W,), jnp.int32),
    pltpu.VMEM((B_PER_W, D_HALF), jnp.float32),
    pltpu.VMEM((B_PER_W, D_HALF), jnp.float32),
]

_sc_half0 = pl.kernel(
    _half0_body,
    out_type=jax.ShapeDtypeStruct((BATCH,), jnp.float32),
    mesh=_MESH,
    scratch_types=_ROWS_SCRATCH + [
        pltpu.VMEM((B_PER_W,), jnp.float32),
    ] + [pltpu.SemaphoreType.DMA] * N_CHUNKS,
    compiler_params=_SC_PARAMS,
)

_sc_half1 = pl.kernel(
    _half1_body,
    out_type=jax.ShapeDtypeStruct((BATCH,), jnp.float32),
    mesh=_MESH,
    scratch_types=_ROWS_SCRATCH + [
        pltpu.VMEM((B_PER_W,), jnp.float32),
        pltpu.VMEM((B_PER_W,), jnp.float32),
        pltpu.VMEM((B_PER_W,), jnp.float32),
        pltpu.SemaphoreType.DMA,
    ] + [pltpu.SemaphoreType.DMA] * N_CHUNKS,
    compiler_params=_SC_PARAMS,
)


@jax.jit
def kernel(product1, product2, product_embedding, product_bias, bias):
    p1 = product1.astype(jnp.int32)
    p2 = product2.astype(jnp.int32)
    bias_flat = product_bias.reshape(EMB_SIZE)
    bias16 = jnp.broadcast_to(bias, (16,))
    emb_a = product_embedding[:, :D_HALF]
    emb_b = product_embedding[:, D_HALF:]
    bsum = _sc_bias(p1, p2, bias_flat, bias16)
    part = _sc_half0(p1, p2, emb_a)
    return _sc_half1(p1, p2, emb_b, part, bsum)


# speedup vs baseline: 1.7942x; 1.7942x over previous
import jax
import jax.numpy as jnp
from jax import lax
from jax.experimental import pallas as pl
from jax.experimental.pallas import tpu as pltpu
from jax.experimental.pallas import tpu_sc as plsc

EMB_SIZE = 100000
EMB_DIM = 64
BATCH = 16384

NUM_CORES = 2
NUM_SUBCORES = 16
NUM_WORKERS = NUM_CORES * NUM_SUBCORES
B_PER_W = BATCH // NUM_WORKERS
IDX_CHUNK = 64
N_CHUNKS = B_PER_W // IDX_CHUNK
GROUPS_PER_CHUNK = IDX_CHUNK // 16


def _dot16(rows1, rows2, base_elem, lane):
    row_idx = base_elem + lane
    d_vec = lane
    acc = jnp.zeros((16,), jnp.float32)
    for _ in range(EMB_DIM):
        a = plsc.load_gather(rows1, [row_idx, d_vec])
        b = plsc.load_gather(rows2, [row_idx, d_vec])
        acc = acc + a * b
        d_vec = (d_vec + 1) % EMB_DIM
    return acc


def _sc_body(p1_hbm, p2_hbm, emb_hbm, bias_hbm, b0_hbm, out_hbm,
             idx1_v, idx2_v, rows1_v, rows2_v, bv1_v, bv2_v, b0_v, out_v,
             sem_b, *sems):
    wid = lax.axis_index("s") * NUM_CORES + lax.axis_index("c")
    base = wid * B_PER_W
    lane = lax.iota(jnp.int32, 16)

    pltpu.sync_copy(p1_hbm.at[pl.ds(base, B_PER_W)], idx1_v)
    pltpu.sync_copy(p2_hbm.at[pl.ds(base, B_PER_W)], idx2_v)
    pltpu.sync_copy(b0_hbm, b0_v)

    copies = []
    for c in range(N_CHUNKS):
        sl = pl.ds(c * IDX_CHUNK, IDX_CHUNK)
        copies.append((
            pltpu.async_copy(emb_hbm.at[idx1_v.at[sl]], rows1_v.at[sl],
                             sems[c]),
            pltpu.async_copy(emb_hbm.at[idx2_v.at[sl]], rows2_v.at[sl],
                             sems[c]),
        ))
    cb1 = pltpu.async_copy(bias_hbm.at[idx1_v], bv1_v, sem_b)
    cb2 = pltpu.async_copy(bias_hbm.at[idx2_v], bv2_v, sem_b)

    for c in range(N_CHUNKS):
        copies[c][0].wait()
        copies[c][1].wait()

        def group_body(g, _):
            e0 = c * IDX_CHUNK + g * 16
            out_v[pl.ds(e0, 16)] = _dot16(rows1_v, rows2_v, e0, lane)
            return 0

        lax.fori_loop(0, GROUPS_PER_CHUNK, group_body, 0)

    cb1.wait()
    cb2.wait()
    b0 = b0_v[...]

    def bias_body(g, _):
        e0 = g * 16
        z = out_v[pl.ds(e0, 16)] + bv1_v[pl.ds(e0, 16)] \
            + bv2_v[pl.ds(e0, 16)] + b0
        out_v[pl.ds(e0, 16)] = 1.0 / (1.0 + jnp.exp(-z))
        return 0

    lax.fori_loop(0, B_PER_W // 16, bias_body, 0)

    pltpu.sync_copy(out_v, out_hbm.at[pl.ds(base, B_PER_W)])


_sc_fused = pl.kernel(
    _sc_body,
    out_type=jax.ShapeDtypeStruct((BATCH,), jnp.float32),
    mesh=plsc.VectorSubcoreMesh(core_axis_name="c", subcore_axis_name="s"),
    scratch_types=[
        pltpu.VMEM((B_PER_W,), jnp.int32),
        pltpu.VMEM((B_PER_W,), jnp.int32),
        pltpu.VMEM((B_PER_W, EMB_DIM), jnp.float32),
        pltpu.VMEM((B_PER_W, EMB_DIM), jnp.float32),
        pltpu.VMEM((B_PER_W,), jnp.float32),
        pltpu.VMEM((B_PER_W,), jnp.float32),
        pltpu.VMEM((16,), jnp.float32),
        pltpu.VMEM((B_PER_W,), jnp.float32),
        pltpu.SemaphoreType.DMA,
    ] + [pltpu.SemaphoreType.DMA] * N_CHUNKS,
    compiler_params=pltpu.CompilerParams(use_tc_tiling_on_sc=False,
                                         needs_layout_passes=False),
)


@jax.jit
def kernel(product1, product2, product_embedding, product_bias, bias):
    p1 = product1.astype(jnp.int32)
    p2 = product2.astype(jnp.int32)
    bias_flat = product_bias.reshape(EMB_SIZE)
    bias16 = jnp.broadcast_to(bias, (16,))
    return _sc_fused(p1, p2, product_embedding, bias_flat, bias16)
